# baseline (speedup 1.0000x reference)
import functools
import math

import jax
import jax.numpy as jnp
from jax import lax
from jax.experimental import pallas as pl
from jax.experimental.pallas import tpu as pltpu
from jax.experimental.pallas import tpu_sc as plsc

_B, _N, _K, _TOPK, _FH = 16, 100, 128, 15, 32
_F32 = jnp.float32
_HALF_LOG_2PI = 0.5 * math.log(2.0 * math.pi)


def _dot(a, b):
    return jax.lax.dot_general(
        a, b, (((a.ndim - 1,), (0,)), ((), ())),
        precision=jax.lax.Precision.DEFAULT,
        preferred_element_type=_F32,
    )


def _prep_kernel(emb_ref, embT_ref, aie_ref, aje_ref,
                 cos_ref, eai_ref, eaj_ref):
    emb = emb_ref[...]
    embT = embT_ref[...]
    nrm_c = jnp.sqrt(jnp.sum(emb * emb, axis=1, keepdims=True))
    nrm_r = jnp.sqrt(jnp.sum(embT * embT, axis=0, keepdims=True))
    cos_ref[...] = _dot(emb, embT) / (nrm_c * nrm_r)
    eai_ref[...] = jnp.sum(emb * aie_ref[...], axis=1, keepdims=True)
    eaj_ref[...] = _dot(aje_ref[...], embT)


_SC_G = _K // 16


def _topk_sc_body(cos_hbm, mask_hbm, cs_v, mask_v):
    wid = lax.axis_index("s") * 2 + lax.axis_index("c")

    @pl.when(wid < _SC_G)
    def _():
        base = wid * 16
        pltpu.sync_copy(cos_hbm.at[:, pl.ds(base, 16)], cs_v)
        ii = lax.iota(jnp.int32, 16)
        zeros = jnp.zeros((16,), _F32)
        for r in range(16):
            for b in range(_K // 16):
                mask_v[r, pl.ds(16 * b, 16)] = zeros

        def step(t, _):
            nc = 4
            m_val = [jnp.full((16,), -3.0, _F32) for _ in range(nc)]
            m_idx = [jnp.zeros((16,), jnp.int32) for _ in range(nc)]
            for t0 in range(_K // nc):
                for p in range(nc):
                    j = nc * t0 + p
                    c = cs_v[j, :]
                    upd = c > m_val[p]
                    m_val[p] = jnp.where(upd, c, m_val[p])
                    m_idx[p] = jnp.where(upd, j, m_idx[p])
            v, ix = m_val[0], m_idx[0]
            for p in range(1, nc):
                take = (m_val[p] > v) | ((m_val[p] == v) & (m_idx[p] < ix))
                v = jnp.where(take, m_val[p], v)
                ix = jnp.where(take, m_idx[p], ix)
            plsc.store_scatter(cs_v, [ix, ii], jnp.full((16,), -3.0, _F32))
            plsc.store_scatter(mask_v, [ii, ix], jnp.ones((16,), _F32))
            return 0

        lax.fori_loop(0, _TOPK, step, 0)
        pltpu.sync_copy(mask_v, mask_hbm.at[pl.ds(base, 16)])


_topk_sc = functools.partial(
    pl.kernel,
    out_type=jax.ShapeDtypeStruct((_K, _K), _F32),
    mesh=plsc.VectorSubcoreMesh(core_axis_name="c", subcore_axis_name="s"),
    scratch_types=[
        pltpu.VMEM((_K, 16), _F32),
        pltpu.VMEM((16, _K), _F32),
    ],
    compiler_params=pltpu.CompilerParams(
        needs_layout_passes=False, use_tc_tiling_on_sc=False),
)(_topk_sc_body)


def _gat_kernel(xt_ref, x_ref, gw_ref, gwT_ref, aix_ref, ajx_ref,
                mask_ref, eai_ref, eaj_ref, bias_ref, gam_ref, bet_ref,
                out_ref):
    xt = xt_ref[0]
    xb = x_ref[0]
    xw = _dot(xt, gw_ref[...])
    xwT = _dot(gwT_ref[...], xb)
    ai = jnp.sum(xw * aix_ref[...], axis=1, keepdims=True) + eai_ref[...]
    aj = _dot(ajx_ref[...], xwT) + eaj_ref[...]
    s = ai + aj
    s = jnp.where(s >= 0, s, 0.2 * s)
    m = mask_ref[...] > 0.5
    smax = jnp.max(jnp.where(m, s, -1e30), axis=1, keepdims=True)
    p = jnp.where(m, jnp.exp(s - smax), 0.0)
    denom = jnp.sum(p, axis=1, keepdims=True)
    attw = p / (denom + 1e-16)
    aggr = _dot(attw, xw) + bias_ref[...]
    h = gam_ref[...] * aggr + bet_ref[...]
    out_ref[0] = jnp.maximum(h, 0.0)


def _temporal_kernel(x_ref, w1_ref, w2_ref, tb_ref, ta_ref, tac_ref, out_ref):
    xb = x_ref[0]
    u = _dot(xb, w1_ref[...]) + tb_ref[...]
    v = _dot(xb, w2_ref[...])
    ta = ta_ref[...]
    sg = jnp.where(ta >= 0, 1.0, -1.0)[None]
    sc = 0.4 * jnp.abs(ta)
    uh = u * sc
    vh = v * sc
    bcol = _dot(v, 0.6 * tac_ref[...])
    blocks = []
    for j0 in list(range(0, 96, 8)) + [92]:
        z = vh[j0:j0 + 8][:, None, :] + uh[None, :, :]
        blocks.append(jnp.sum(jnp.abs(z) * sg, axis=2))
    eT = jnp.concatenate(blocks[:12] + [blocks[12][4:]], axis=0) + bcol
    emax = jnp.max(eT, axis=0, keepdims=True)
    pe = jnp.exp(eT - emax)
    attnT = pe / jnp.sum(pe, axis=0, keepdims=True)
    out_ref[0] = jax.nn.sigmoid(jax.lax.dot_general(
        attnT, xb, (((0,), (0,)), ((), ())),
        precision=jax.lax.Precision.DEFAULT,
        preferred_element_type=_F32))


def _head_kernel(hf_ref, ht_ref, x_ref, viwf_ref, viwt_ref, vib_ref,
                 vew1_ref, veb1_ref, vewmu_ref, vebmu_ref,
                 vdw1_ref, vdb1_ref, vdwo_ref, vdbo_ref, scal_ref,
                 flow_ref, vae_ref):
    hf = hf_ref[0]
    ht = ht_ref[0]
    xb = x_ref[0]

    vin = _dot(hf, viwf_ref[...]) + _dot(ht, viwt_ref[...]) + vib_ref[...]
    he = jnp.tanh(_dot(vin, vew1_ref[...]) + veb1_ref[...])
    mu = _dot(he, vewmu_ref[...]) + vebmu_ref[...]
    hd = jnp.tanh(_dot(mu, vdw1_ref[...]) + vdb1_ref[...])
    recon = _dot(hd, vdwo_ref[...]) + vdbo_ref[...]
    vae_ref[0] = -0.5 * (xb - recon) ** 2 - _HALF_LOG_2PI

    ic = jax.lax.broadcasted_iota(jnp.int32, (_K, _K), 0)
    im = jax.lax.broadcasted_iota(jnp.int32, (_K, _K), 1)
    lo = im < 64
    s0f = jnp.where(lo & (ic == 2 * im), 1.0, 0.0)
    s0t = jnp.where(~lo & (ic == 2 * im - 128), 1.0, 0.0)
    s1f = jnp.where(lo & (ic == 2 * im + 1), 1.0, 0.0)
    s1t = jnp.where(~lo & (ic == 2 * im - 127), 1.0, 0.0)
    E = _dot(hf, s0f) + _dot(ht, s0t)
    O = _dot(hf, s1f) + _dot(ht, s1t)

    accm = jnp.zeros((_N, _K), _F32)
    accl = jnp.zeros((_N, _K), _F32)
    for h in range(_FH):
        t = jnp.tanh(scal_ref[0, h] * E + scal_ref[1, h] * O
                     + scal_ref[2, h])
        accm = accm + scal_ref[3, h] * t
        accl = accl + scal_ref[4, h] * t
    m_ = accm + scal_ref[5, 0]
    loga = accl + scal_ref[5, 1]
    exp_lg = scal_ref[5, 4]
    btf = scal_ref[5, 3]
    cterm = scal_ref[5, 5]
    uu = (xb - m_) * jnp.exp(-loga)
    u2 = exp_lg * uu + btf
    lp = -0.5 * u2 * u2 - loga + cterm
    flow_ref[0] = jnp.mean(lp, axis=0, keepdims=True)


def kernel(x, params, train):
    p = params
    emb = p["embedding"]
    embT = jnp.transpose(emb)
    xt = jnp.transpose(x, (0, 2, 1))

    ai = p["gat_att_i"]
    aj = p["gat_att_j"]
    aix = ai[:_N].reshape(1, _N)
    aie = ai[_N:].reshape(1, _N)
    ajx = aj[:_N].reshape(1, _N)
    aje = aj[_N:].reshape(1, _N)

    cos, eai, eaj = pl.pallas_call(
        _prep_kernel,
        out_shape=(
            jax.ShapeDtypeStruct((_K, _K), _F32),
            jax.ShapeDtypeStruct((_K, 1), _F32),
            jax.ShapeDtypeStruct((1, _K), _F32),
        ),
    )(emb, embT, aie, aje)
    mask = _topk_sc(cos)

    gw = p["gat_W"]
    gwT = jnp.transpose(gw)
    row = lambda v: v.reshape(1, -1)
    b_spec = lambda r, c: pl.BlockSpec((1, r, c), lambda b: (b, 0, 0))
    w_spec = lambda r, c: pl.BlockSpec((r, c), lambda b: (0, 0))

    tw = p["t_W"]
    h_time = pl.pallas_call(
        _temporal_kernel,
        grid=(_B,),
        in_specs=[
            b_spec(_N, _K),
            w_spec(_K, 2 * _K), w_spec(_K, 2 * _K),
            w_spec(1, 2 * _K), w_spec(1, 2 * _K), w_spec(2 * _K, 1),
        ],
        out_specs=b_spec(_N, _K),
        out_shape=jax.ShapeDtypeStruct((_B, _N, _K), _F32),
    )(x, tw[:_K], tw[_K:], row(p["t_b"]), row(p["t_a"]),
      p["t_a"].reshape(2 * _K, 1))

    h_feat = pl.pallas_call(
        _gat_kernel,
        grid=(_B,),
        in_specs=[
            b_spec(_K, _N), b_spec(_N, _K),
            w_spec(_N, _N), w_spec(_N, _N),
            w_spec(1, _N), w_spec(1, _N),
            w_spec(_K, _K), w_spec(_K, 1), w_spec(1, _K),
            w_spec(1, _N), w_spec(1, _N), w_spec(1, _N),
        ],
        out_specs=b_spec(_K, _N),
        out_shape=jax.ShapeDtypeStruct((_B, _K, _N), _F32),
    )(xt, x, gw, gwT, aix, ajx, mask, eai, eaj,
      row(p["gat_bias"]), row(p["gat_bn_gamma"]), row(p["gat_bn_beta"]))

    hfT = jnp.transpose(h_feat, (0, 2, 1))

    lg = p["bnf_log_gamma"][0]
    pad = jnp.zeros((26,), _F32)
    scal = jnp.stack([
        p["made_Wc"][0], p["made_Wc"][1], p["made_b1"],
        p["made_Wo"][:, 0], p["made_Wo"][:, 1],
        jnp.concatenate([
            p["made_bo"][:1], p["made_bo"][1:],
            lg[None], p["bnf_beta"], jnp.exp(lg)[None],
            (lg - 0.5 * math.log(1.0 + 1e-5) - _HALF_LOG_2PI)[None], pad]),
    ])

    vi = p["vi_W"]
    flow3, vae_lp = pl.pallas_call(
        _head_kernel,
        grid=(_B,),
        in_specs=[
            b_spec(_N, _K), b_spec(_N, _K), b_spec(_N, _K),
            w_spec(_K, _K), w_spec(_K, _K), w_spec(1, _K),
            w_spec(_K, 2 * _K), w_spec(1, 2 * _K),
            w_spec(2 * _K, _K), w_spec(1, _K),
            w_spec(_K, 2 * _K), w_spec(1, 2 * _K),
            w_spec(2 * _K, _K), w_spec(1, _K),
            pl.BlockSpec((6, _FH), lambda b: (0, 0),
                         memory_space=pltpu.SMEM),
        ],
        out_specs=(
            pl.BlockSpec((1, 1, _K), lambda b: (b, 0, 0)),
            b_spec(_N, _K),
        ),
        out_shape=(
            jax.ShapeDtypeStruct((_B, 1, _K), _F32),
            jax.ShapeDtypeStruct((_B, _N, _K), _F32),
        ),
    )(hfT, h_time, x,
      vi[:_K], vi[_K:], row(p["vi_b"]),
      p["ve_W1"], row(p["ve_b1"]), p["ve_Wmu"], row(p["ve_bmu"]),
      p["vd_W1"], row(p["vd_b1"]), p["vd_Wo"], row(p["vd_bo"]), scal)

    return flow3.reshape(_B, _K), vae_lp


# speedup vs baseline: 1.1276x; 1.1276x over previous
import functools
import math

import jax
import jax.numpy as jnp
from jax import lax
from jax.experimental import pallas as pl
from jax.experimental.pallas import tpu as pltpu
from jax.experimental.pallas import tpu_sc as plsc

_B, _N, _K, _TOPK, _FH = 16, 100, 128, 15, 32
_F32 = jnp.float32
_HALF_LOG_2PI = 0.5 * math.log(2.0 * math.pi)


def _dot(a, b):
    return jax.lax.dot_general(
        a, b, (((a.ndim - 1,), (0,)), ((), ())),
        precision=jax.lax.Precision.DEFAULT,
        preferred_element_type=_F32,
    )


def _prep_kernel(emb_ref, embT_ref, aie_ref, aje_ref,
                 cos_ref, eai_ref, eaj_ref):
    emb = emb_ref[...]
    embT = embT_ref[...]
    nrm_c = jnp.sqrt(jnp.sum(emb * emb, axis=1, keepdims=True))
    nrm_r = jnp.sqrt(jnp.sum(embT * embT, axis=0, keepdims=True))
    cos_ref[...] = _dot(emb, embT) / (nrm_c * nrm_r)
    eai_ref[...] = jnp.sum(emb * aie_ref[...], axis=1, keepdims=True)
    eaj_ref[...] = _dot(aje_ref[...], embT)


_SC_G = _K // 16


def _topk_sc_body(cos_hbm, mask_hbm, cs_v, mask_v):
    wid = lax.axis_index("s") * 2 + lax.axis_index("c")

    @pl.when(wid < _SC_G)
    def _():
        base = wid * 16
        pltpu.sync_copy(cos_hbm.at[:, pl.ds(base, 16)], cs_v)
        ii = lax.iota(jnp.int32, 16)
        zeros = jnp.zeros((16,), _F32)
        for r in range(16):
            for b in range(_K // 16):
                mask_v[r, pl.ds(16 * b, 16)] = zeros

        def step(t, _):
            nc = 4
            m_val = [jnp.full((16,), -3.0, _F32) for _ in range(nc)]
            m_idx = [jnp.zeros((16,), jnp.int32) for _ in range(nc)]
            for t0 in range(_K // nc):
                for p in range(nc):
                    j = nc * t0 + p
                    c = cs_v[j, :]
                    upd = c > m_val[p]
                    m_val[p] = jnp.where(upd, c, m_val[p])
                    m_idx[p] = jnp.where(upd, j, m_idx[p])
            v, ix = m_val[0], m_idx[0]
            for p in range(1, nc):
                take = (m_val[p] > v) | ((m_val[p] == v) & (m_idx[p] < ix))
                v = jnp.where(take, m_val[p], v)
                ix = jnp.where(take, m_idx[p], ix)
            plsc.store_scatter(cs_v, [ix, ii], jnp.full((16,), -3.0, _F32))
            plsc.store_scatter(mask_v, [ii, ix], jnp.ones((16,), _F32))
            return 0

        lax.fori_loop(0, _TOPK, step, 0)
        pltpu.sync_copy(mask_v, mask_hbm.at[pl.ds(base, 16)])


_topk_sc = functools.partial(
    pl.kernel,
    out_type=jax.ShapeDtypeStruct((_K, _K), _F32),
    mesh=plsc.VectorSubcoreMesh(core_axis_name="c", subcore_axis_name="s"),
    scratch_types=[
        pltpu.VMEM((_K, 16), _F32),
        pltpu.VMEM((16, _K), _F32),
    ],
    compiler_params=pltpu.CompilerParams(
        needs_layout_passes=False, use_tc_tiling_on_sc=False),
)(_topk_sc_body)


def _dotT(a, b):
    return jax.lax.dot_general(
        a, b, (((0,), (0,)), ((), ())),
        precision=jax.lax.Precision.DEFAULT,
        preferred_element_type=_F32,
    )


def _fused_kernel(xt_ref, x_ref, gw_ref, gwT_ref, aix_ref, ajx_ref,
                  mask_ref, eai_ref, eaj_ref, bias_ref, gam_ref, bet_ref,
                  w1_ref, w2_ref, tb_ref, ta_ref, tac_ref,
                  viwf_ref, viwt_ref, vib_ref,
                  vew1_ref, veb1_ref, vewmu_ref, vebmu_ref,
                  vdw1_ref, vdb1_ref, vdwo_ref, vdbo_ref, scal_ref,
                  flow_ref, vae_ref):
    xt = xt_ref[0]
    xb = x_ref[0]

    xw = _dot(xt, gw_ref[...])
    xwT = _dot(gwT_ref[...], xb)
    ai = jnp.sum(xw * aix_ref[...], axis=1, keepdims=True) + eai_ref[...]
    aj = _dot(ajx_ref[...], xwT) + eaj_ref[...]
    s = ai + aj
    s = jnp.where(s >= 0, s, 0.2 * s)
    m = mask_ref[...] > 0.5
    smax = jnp.max(jnp.where(m, s, -1e30), axis=1, keepdims=True)
    p = jnp.where(m, jnp.exp(s - smax), 0.0)
    denom = jnp.sum(p, axis=1, keepdims=True)
    attw = p / (denom + 1e-16)
    aggr = _dot(attw, xw) + bias_ref[...]
    hf = jnp.maximum(gam_ref[...] * aggr + bet_ref[...], 0.0)

    u = _dot(xb, w1_ref[...]) + tb_ref[...]
    v = _dot(xb, w2_ref[...])
    ta = ta_ref[...]
    sg = jnp.where(ta >= 0, 1.0, -1.0)[None]
    scl = 0.4 * jnp.abs(ta)
    uh = u * scl
    vh = v * scl
    bcol = _dot(v, 0.6 * tac_ref[...])
    blocks = []
    for j0 in list(range(0, 96, 8)) + [92]:
        z = vh[j0:j0 + 8][:, None, :] + uh[None, :, :]
        blocks.append(jnp.sum(jnp.abs(z) * sg, axis=2))
    eT = jnp.concatenate(blocks[:12] + [blocks[12][4:]], axis=0) + bcol
    emax = jnp.max(eT, axis=0, keepdims=True)
    pe = jnp.exp(eT - emax)
    attnT = pe / jnp.sum(pe, axis=0, keepdims=True)
    ht = jax.nn.sigmoid(_dotT(attnT, xb))

    vin = _dotT(hf, viwf_ref[...]) + _dot(ht, viwt_ref[...]) + vib_ref[...]
    he = jnp.tanh(_dot(vin, vew1_ref[...]) + veb1_ref[...])
    mu = _dot(he, vewmu_ref[...]) + vebmu_ref[...]
    hd = jnp.tanh(_dot(mu, vdw1_ref[...]) + vdb1_ref[...])
    recon = _dot(hd, vdwo_ref[...]) + vdbo_ref[...]
    vae_ref[0] = -0.5 * (xb - recon) ** 2 - _HALF_LOG_2PI

    ic = jax.lax.broadcasted_iota(jnp.int32, (_K, _K), 0)
    im = jax.lax.broadcasted_iota(jnp.int32, (_K, _K), 1)
    lo = im < 64
    s0f = jnp.where(lo & (ic == 2 * im), 1.0, 0.0)
    s0t = jnp.where(~lo & (ic == 2 * im - 128), 1.0, 0.0)
    s1f = jnp.where(lo & (ic == 2 * im + 1), 1.0, 0.0)
    s1t = jnp.where(~lo & (ic == 2 * im - 127), 1.0, 0.0)
    E = _dotT(hf, s0f) + _dot(ht, s0t)
    O = _dotT(hf, s1f) + _dot(ht, s1t)

    accm = jnp.zeros((_N, _K), _F32)
    accl = jnp.zeros((_N, _K), _F32)
    for h in range(_FH):
        t = jnp.tanh(scal_ref[0, h] * E + scal_ref[1, h] * O
                     + scal_ref[2, h])
        accm = accm + scal_ref[3, h] * t
        accl = accl + scal_ref[4, h] * t
    m_ = accm + scal_ref[5, 0]
    loga = accl + scal_ref[5, 1]
    exp_lg = scal_ref[5, 4]
    btf = scal_ref[5, 3]
    cterm = scal_ref[5, 5]
    uu = (xb - m_) * jnp.exp(-loga)
    u2 = exp_lg * uu + btf
    lp = -0.5 * u2 * u2 - loga + cterm
    flow_ref[0] = jnp.mean(lp, axis=0, keepdims=True)


def kernel(x, params, train):
    p = params
    emb = p["embedding"]
    embT = jnp.transpose(emb)
    xt = jnp.transpose(x, (0, 2, 1))

    ai = p["gat_att_i"]
    aj = p["gat_att_j"]
    aix = ai[:_N].reshape(1, _N)
    aie = ai[_N:].reshape(1, _N)
    ajx = aj[:_N].reshape(1, _N)
    aje = aj[_N:].reshape(1, _N)

    cos, eai, eaj = pl.pallas_call(
        _prep_kernel,
        out_shape=(
            jax.ShapeDtypeStruct((_K, _K), _F32),
            jax.ShapeDtypeStruct((_K, 1), _F32),
            jax.ShapeDtypeStruct((1, _K), _F32),
        ),
    )(emb, embT, aie, aje)
    mask = _topk_sc(cos)

    gw = p["gat_W"]
    gwT = jnp.transpose(gw)
    row = lambda v: v.reshape(1, -1)
    b_spec = lambda r, c: pl.BlockSpec((1, r, c), lambda b: (b, 0, 0))
    w_spec = lambda r, c: pl.BlockSpec((r, c), lambda b: (0, 0))

    lg = p["bnf_log_gamma"][0]
    pad = jnp.zeros((26,), _F32)
    scal = jnp.stack([
        p["made_Wc"][0], p["made_Wc"][1], p["made_b1"],
        p["made_Wo"][:, 0], p["made_Wo"][:, 1],
        jnp.concatenate([
            p["made_bo"][:1], p["made_bo"][1:],
            lg[None], p["bnf_beta"], jnp.exp(lg)[None],
            (lg - 0.5 * math.log(1.0 + 1e-5) - _HALF_LOG_2PI)[None], pad]),
    ])

    tw = p["t_W"]
    vi = p["vi_W"]
    flow3, vae_lp = pl.pallas_call(
        _fused_kernel,
        grid=(_B,),
        in_specs=[
            b_spec(_K, _N), b_spec(_N, _K),
            w_spec(_N, _N), w_spec(_N, _N),
            w_spec(1, _N), w_spec(1, _N),
            w_spec(_K, _K), w_spec(_K, 1), w_spec(1, _K),
            w_spec(1, _N), w_spec(1, _N), w_spec(1, _N),
            w_spec(_K, 2 * _K), w_spec(_K, 2 * _K),
            w_spec(1, 2 * _K), w_spec(1, 2 * _K), w_spec(2 * _K, 1),
            w_spec(_K, _K), w_spec(_K, _K), w_spec(1, _K),
            w_spec(_K, 2 * _K), w_spec(1, 2 * _K),
            w_spec(2 * _K, _K), w_spec(1, _K),
            w_spec(_K, 2 * _K), w_spec(1, 2 * _K),
            w_spec(2 * _K, _K), w_spec(1, _K),
            pl.BlockSpec((6, _FH), lambda b: (0, 0),
                         memory_space=pltpu.SMEM),
        ],
        out_specs=(
            pl.BlockSpec((1, 1, _K), lambda b: (b, 0, 0)),
            b_spec(_N, _K),
        ),
        out_shape=(
            jax.ShapeDtypeStruct((_B, 1, _K), _F32),
            jax.ShapeDtypeStruct((_B, _N, _K), _F32),
        ),
    )(xt, x, gw, gwT, aix, ajx, mask, eai, eaj,
      row(p["gat_bias"]), row(p["gat_bn_gamma"]), row(p["gat_bn_beta"]),
      tw[:_K], tw[_K:], row(p["t_b"]), row(p["t_a"]),
      p["t_a"].reshape(2 * _K, 1),
      vi[:_K], vi[_K:], row(p["vi_b"]),
      p["ve_W1"], row(p["ve_b1"]), p["ve_Wmu"], row(p["ve_bmu"]),
      p["vd_W1"], row(p["vd_b1"]), p["vd_Wo"], row(p["vd_bo"]), scal)

    return flow3.reshape(_B, _K), vae_lp
